# baseline (speedup 1.0000x reference)
import functools

import jax
import jax.numpy as jnp
from jax import lax
from jax.experimental import pallas as pl
from jax.experimental.pallas import tpu as pltpu
from jax.experimental.pallas import tpu_sc as plsc

_N = 10000
_D = 128
_E = 320000
_NROWS = 10112
_NC = 2
_NS = 16
_NW = _NC * _NS
_CHUNK = 128
_G = _E // _CHUNK
_G0 = _G // 2
_RPT = _NROWS // _NS
_SLOPE = 0.2


def _dense_body(x_ref, wt_ref, b_ref, a2_ref, tab_ref, w1_ref):
    x = x_ref[...]
    h = jnp.dot(x, wt_ref[...], preferred_element_type=jnp.float32) + b_ref[...]
    x12 = jnp.where(h > 0, h, _SLOPE * h)
    e2 = jnp.dot(x12, a2_ref[...], preferred_element_type=jnp.float32)
    w = jnp.exp(e2 - jnp.max(e2))
    tab_ref[...] = h * w
    w1_ref[...] = jnp.reshape(w, (_N,))


def _stripe_copy(src_fn, dst_fn):
    full, rem = _RPT // _CHUNK, _RPT % _CHUNK
    for k in range(full):
        pltpu.sync_copy(src_fn(k * _CHUNK, _CHUNK), dst_fn(k * _CHUNK, _CHUNK))
    if rem:
        pltpu.sync_copy(src_fn(full * _CHUNK, rem), dst_fn(full * _CHUNK, rem))


def _sc_scatter(ei_hbm, table_hbm, w_hbm, zeros_hbm, zeros1_hbm,
                out_hbm, den_hbm,
                acc, den, col_a, col_b, row_a, row_b, srow_a, srow_b,
                rows_a, rows_b, wv_a, wv_b,
                sem_a, sem_b, isem_a, isem_b, ssem_a, ssem_b):
    c = lax.axis_index("c")
    s = lax.axis_index("s")
    r0 = s * _RPT
    pltpu.sync_copy(zeros_hbm, rows_a)
    pltpu.sync_copy(zeros1_hbm, wv_a)
    _stripe_copy(lambda o, n: rows_a.at[pl.ds(0, n)],
                 lambda o, n: acc.at[pl.ds(r0 + o, n)])
    _stripe_copy(lambda o, n: wv_a.at[pl.ds(0, n)],
                 lambda o, n: den.at[pl.ds(r0 + o, n)])
    plsc.subcore_barrier()

    pairs0 = _G0 // 2
    pairs1 = _G // 2 - pairs0
    pc = pairs0 + c * (pairs1 - pairs0)
    q, r = pc // _NS, pc % _NS
    extra = jnp.where(s < r, 1, 0).astype(jnp.int32)
    np_ = q + extra
    start = 2 * (c * pairs0 + s * q + jnp.minimum(s, r))

    def gather(cbuf, buf, wv, sem):
        pltpu.async_copy(table_hbm.at[cbuf], buf, sem)
        pltpu.async_copy(w_hbm.at[cbuf], wv, sem)

    def wait_gather(cbuf, buf, wv, sem):
        pltpu.make_async_copy(table_hbm.at[cbuf], buf, sem).wait()
        pltpu.make_async_copy(w_hbm.at[cbuf], wv, sem).wait()

    def scat(rbuf, buf, wv, ssem):
        pltpu.async_copy(buf, acc.at[rbuf], ssem, add=True)
        pltpu.async_copy(wv, den.at[rbuf], ssem, add=True)

    def wait_scat(rbuf, buf, wv, ssem):
        pltpu.make_async_copy(buf, acc.at[rbuf], ssem).wait()
        pltpu.make_async_copy(wv, den.at[rbuf], ssem).wait()

    def copy_idx(src, dst):
        for k in range(_CHUNK // 16):
            dst[pl.ds(k * 16, 16)] = src[pl.ds(k * 16, 16)]

    def fetch_idx(j, cbuf, rbuf, isem):
        base = (start + j) * _CHUNK
        pltpu.async_copy(ei_hbm.at[1, pl.ds(base, _CHUNK)], cbuf, isem)
        pltpu.async_copy(ei_hbm.at[0, pl.ds(base, _CHUNK)], rbuf, isem)

    def wait_idx(j, cbuf, rbuf, isem):
        base = (start + j) * _CHUNK
        pltpu.make_async_copy(
            ei_hbm.at[1, pl.ds(base, _CHUNK)], cbuf, isem).wait()
        pltpu.make_async_copy(
            ei_hbm.at[0, pl.ds(base, _CHUNK)], rbuf, isem).wait()

    fetch_idx(0, col_a, row_a, isem_a)
    wait_idx(0, col_a, row_a, isem_a)
    gather(col_a, rows_a, wv_a, sem_a)
    fetch_idx(1, col_b, row_b, isem_b)
    wait_idx(1, col_b, row_b, isem_b)

    def body(i, carry):
        ja = 2 * i
        gather(col_b, rows_b, wv_b, sem_b)
        wait_gather(col_a, rows_a, wv_a, sem_a)
        copy_idx(row_a, srow_a)
        fetch_idx(ja + 2, col_a, row_a, isem_a)
        scat(srow_a, rows_a, wv_a, ssem_a)
        wait_idx(ja + 2, col_a, row_a, isem_a)
        wait_scat(srow_a, rows_a, wv_a, ssem_a)
        gather(col_a, rows_a, wv_a, sem_a)
        wait_gather(col_b, rows_b, wv_b, sem_b)
        copy_idx(row_b, srow_b)
        fetch_idx(ja + 3, col_b, row_b, isem_b)
        scat(srow_b, rows_b, wv_b, ssem_b)
        wait_idx(ja + 3, col_b, row_b, isem_b)
        wait_scat(srow_b, rows_b, wv_b, ssem_b)
        return carry

    lax.fori_loop(0, np_ - 1, body, 0)
    last = 2 * np_ - 2
    gather(col_b, rows_b, wv_b, sem_b)
    wait_gather(col_a, rows_a, wv_a, sem_a)
    scat(row_a, rows_a, wv_a, ssem_a)
    wait_gather(col_b, rows_b, wv_b, sem_b)
    scat(row_b, rows_b, wv_b, ssem_b)
    wait_scat(row_a, rows_a, wv_a, ssem_a)
    wait_scat(row_b, rows_b, wv_b, ssem_b)
    plsc.subcore_barrier()
    _stripe_copy(lambda o, n: acc.at[pl.ds(r0 + o, n)],
                 lambda o, n: out_hbm.at[c, pl.ds(r0 + o, n)])

    def den_out(o, n):
        pltpu.sync_copy(den.at[pl.ds(r0 + o, n)], wv_a.at[pl.ds(0, n)])
        pltpu.sync_copy(wv_a.at[pl.ds(0, n)],
                        den_hbm.at[pl.ds(c * _NROWS + r0 + o, n)])

    for k in range(_RPT // _CHUNK):
        den_out(k * _CHUNK, _CHUNK)
    if _RPT % _CHUNK:
        den_out((_RPT // _CHUNK) * _CHUNK, _RPT % _CHUNK)


def _norm_body(a0_ref, a1_ref, t_ref, d_ref, o_ref):
    num = a0_ref[0] + a1_ref[0] + t_ref[...]
    o_ref[...] = num / d_ref[...]


def kernel(x, edge_index, W1_w, W1_b, a1, a2):
    del a1
    table, w = pl.pallas_call(
        _dense_body,
        out_shape=[jax.ShapeDtypeStruct((_N, _D), jnp.float32),
                   jax.ShapeDtypeStruct((_N,), jnp.float32)],
    )(x, W1_w.T, W1_b.reshape(1, _D), a2.reshape(_D, 1))

    ei = edge_index.astype(jnp.int32)
    zeros_tile = jnp.zeros((_CHUNK, _D), jnp.float32)
    zeros_one = jnp.zeros((_CHUNK,), jnp.float32)

    scatter = functools.partial(
        pl.kernel,
        out_type=[jax.ShapeDtypeStruct((_NC, _NROWS, _D), jnp.float32),
                  jax.ShapeDtypeStruct((_NC * _NROWS,), jnp.float32)],
        mesh=plsc.VectorSubcoreMesh(core_axis_name="c", subcore_axis_name="s"),
        compiler_params=pltpu.CompilerParams(use_tc_tiling_on_sc=True),
        scratch_types=[
            pltpu.VMEM_SHARED((_NROWS, _D), jnp.float32),
            pltpu.VMEM_SHARED((_NROWS,), jnp.float32),
            pltpu.VMEM((_CHUNK,), jnp.int32),
            pltpu.VMEM((_CHUNK,), jnp.int32),
            pltpu.VMEM((_CHUNK,), jnp.int32),
            pltpu.VMEM((_CHUNK,), jnp.int32),
            pltpu.VMEM((_CHUNK,), jnp.int32),
            pltpu.VMEM((_CHUNK,), jnp.int32),
            pltpu.VMEM((_CHUNK, _D), jnp.float32),
            pltpu.VMEM((_CHUNK, _D), jnp.float32),
            pltpu.VMEM((_CHUNK,), jnp.float32),
            pltpu.VMEM((_CHUNK,), jnp.float32),
            pltpu.SemaphoreType.DMA,
            pltpu.SemaphoreType.DMA,
            pltpu.SemaphoreType.DMA,
            pltpu.SemaphoreType.DMA,
            pltpu.SemaphoreType.DMA,
            pltpu.SemaphoreType.DMA,
        ],
    )(_sc_scatter)
    acc, denp = scatter(ei, table, w, zeros_tile, zeros_one)
    dsum = (denp[:_N] + denp[_NROWS:_NROWS + _N] + w).reshape(_N, 1)

    nb = _N // 10
    out = pl.pallas_call(
        _norm_body,
        grid=(10,),
        in_specs=[
            pl.BlockSpec((1, nb, _D), lambda i: (0, i, 0)),
            pl.BlockSpec((1, nb, _D), lambda i: (1, i, 0)),
            pl.BlockSpec((nb, _D), lambda i: (i, 0)),
            pl.BlockSpec((nb, 1), lambda i: (i, 0)),
        ],
        out_specs=pl.BlockSpec((nb, _D), lambda i: (i, 0)),
        out_shape=jax.ShapeDtypeStruct((_N, _D), jnp.float32),
    )(acc, acc, table, dsum)
    return out


# speedup vs baseline: 1.0026x; 1.0026x over previous
import functools

import jax
import jax.numpy as jnp
from jax import lax
from jax.experimental import pallas as pl
from jax.experimental.pallas import tpu as pltpu
from jax.experimental.pallas import tpu_sc as plsc

_N = 10000
_D = 128
_E = 320000
_NROWS = 10112
_NC = 2
_NS = 16
_CHUNK = 128
_G = _E // _CHUNK
_G0 = _G // 2
_RPT = _NROWS // _NS
_SLOPE = 0.2


def _dense_body(x_ref, wt_ref, b_ref, a2_ref, tab_ref, w1_ref):
    x = x_ref[...]
    h = jnp.dot(x, wt_ref[...], preferred_element_type=jnp.float32) + b_ref[...]
    x12 = jnp.where(h > 0, h, _SLOPE * h)
    e2 = jnp.dot(x12, a2_ref[...], preferred_element_type=jnp.float32)
    w = jnp.exp(e2 - jnp.max(e2))
    tab_ref[...] = h * w
    w1_ref[...] = jnp.reshape(w, (_N,))


def _stripe_copy(src_fn, dst_fn):
    full, rem = _RPT // _CHUNK, _RPT % _CHUNK
    for k in range(full):
        pltpu.sync_copy(src_fn(k * _CHUNK, _CHUNK), dst_fn(k * _CHUNK, _CHUNK))
    if rem:
        pltpu.sync_copy(src_fn(full * _CHUNK, rem), dst_fn(full * _CHUNK, rem))


def _sc_scatter(ei_hbm, table_hbm, w_hbm, zeros_hbm, zeros1_hbm,
                out_hbm, den_hbm,
                acc, den, col_a, col_b, row_a, row_b, srow_a, srow_b,
                rows_a, rows_b, wv_a, wv_b,
                sem_a, sem_b, isem_a, isem_b, ssem_a, ssem_b):
    c = lax.axis_index("c")
    s = lax.axis_index("s")
    r0 = s * _RPT
    pltpu.sync_copy(zeros_hbm, rows_a)
    pltpu.sync_copy(zeros1_hbm, wv_a)
    _stripe_copy(lambda o, n: rows_a.at[pl.ds(0, n)],
                 lambda o, n: acc.at[pl.ds(r0 + o, n)])
    _stripe_copy(lambda o, n: wv_a.at[pl.ds(0, n)],
                 lambda o, n: den.at[pl.ds(r0 + o, n)])
    plsc.subcore_barrier()

    pairs0 = _G0 // 2
    pairs1 = _G // 2 - pairs0
    pc = pairs0 + c * (pairs1 - pairs0)
    q, r = pc // _NS, pc % _NS
    extra = jnp.where(s < r, 1, 0).astype(jnp.int32)
    np_ = q + extra
    start = 2 * (c * pairs0 + s * q + jnp.minimum(s, r))

    def gather(cbuf, buf, wv, sem):
        pltpu.async_copy(table_hbm.at[cbuf], buf, sem)
        pltpu.async_copy(w_hbm.at[cbuf], wv, sem)

    def wait_gather(cbuf, buf, wv, sem):
        pltpu.make_async_copy(table_hbm.at[cbuf], buf, sem).wait()
        pltpu.make_async_copy(w_hbm.at[cbuf], wv, sem).wait()

    def scat(rbuf, buf, wv, ssem):
        pltpu.async_copy(buf, acc.at[rbuf], ssem, add=True)
        pltpu.async_copy(wv, den.at[rbuf], ssem, add=True)

    def wait_scat(rbuf, buf, wv, ssem):
        pltpu.make_async_copy(buf, acc.at[rbuf], ssem).wait()
        pltpu.make_async_copy(wv, den.at[rbuf], ssem).wait()

    def copy_idx(src, dst):
        for k in range(_CHUNK // 16):
            dst[pl.ds(k * 16, 16)] = src[pl.ds(k * 16, 16)]

    def fetch_idx(j, cbuf, rbuf, isem):
        base = (start + j) * _CHUNK
        pltpu.async_copy(ei_hbm.at[1, pl.ds(base, _CHUNK)], cbuf, isem)
        pltpu.async_copy(ei_hbm.at[0, pl.ds(base, _CHUNK)], rbuf, isem)

    def wait_idx(j, cbuf, rbuf, isem):
        base = (start + j) * _CHUNK
        pltpu.make_async_copy(
            ei_hbm.at[1, pl.ds(base, _CHUNK)], cbuf, isem).wait()
        pltpu.make_async_copy(
            ei_hbm.at[0, pl.ds(base, _CHUNK)], rbuf, isem).wait()

    fetch_idx(0, col_a, row_a, isem_a)
    wait_idx(0, col_a, row_a, isem_a)
    gather(col_a, rows_a, wv_a, sem_a)
    fetch_idx(1, col_b, row_b, isem_b)
    wait_idx(1, col_b, row_b, isem_b)

    def body(i, carry):
        ja = 2 * i
        gather(col_b, rows_b, wv_b, sem_b)
        wait_gather(col_a, rows_a, wv_a, sem_a)
        copy_idx(row_a, srow_a)
        fetch_idx(ja + 2, col_a, row_a, isem_a)
        scat(srow_a, rows_a, wv_a, ssem_a)
        wait_idx(ja + 2, col_a, row_a, isem_a)
        wait_scat(srow_a, rows_a, wv_a, ssem_a)
        gather(col_a, rows_a, wv_a, sem_a)
        wait_gather(col_b, rows_b, wv_b, sem_b)
        copy_idx(row_b, srow_b)
        fetch_idx(ja + 3, col_b, row_b, isem_b)
        scat(srow_b, rows_b, wv_b, ssem_b)
        wait_idx(ja + 3, col_b, row_b, isem_b)
        wait_scat(srow_b, rows_b, wv_b, ssem_b)
        return carry

    lax.fori_loop(0, np_ - 1, body, 0)
    gather(col_b, rows_b, wv_b, sem_b)
    wait_gather(col_a, rows_a, wv_a, sem_a)
    scat(row_a, rows_a, wv_a, ssem_a)
    wait_gather(col_b, rows_b, wv_b, sem_b)
    scat(row_b, rows_b, wv_b, ssem_b)
    wait_scat(row_a, rows_a, wv_a, ssem_a)
    wait_scat(row_b, rows_b, wv_b, ssem_b)
    plsc.subcore_barrier()
    _stripe_copy(lambda o, n: acc.at[pl.ds(r0 + o, n)],
                 lambda o, n: out_hbm.at[c, pl.ds(r0 + o, n)])

    def den_out(o, n):
        pltpu.sync_copy(den.at[pl.ds(r0 + o, n)], wv_a.at[pl.ds(0, n)])
        pltpu.sync_copy(wv_a.at[pl.ds(0, n)],
                        den_hbm.at[pl.ds(c * _NROWS + r0 + o, n)])

    for k in range(_RPT // _CHUNK):
        den_out(k * _CHUNK, _CHUNK)
    if _RPT % _CHUNK:
        den_out((_RPT // _CHUNK) * _CHUNK, _RPT % _CHUNK)


def _norm_body(a0_ref, a1_ref, t_ref, d_ref, o_ref):
    num = a0_ref[0] + a1_ref[0] + t_ref[...]
    o_ref[...] = num / d_ref[...]


def kernel(x, edge_index, W1_w, W1_b, a1, a2):
    del a1
    table, w = pl.pallas_call(
        _dense_body,
        out_shape=[jax.ShapeDtypeStruct((_N, _D), jnp.float32),
                   jax.ShapeDtypeStruct((_N,), jnp.float32)],
    )(x, W1_w.T, W1_b.reshape(1, _D), a2.reshape(_D, 1))

    ei = edge_index.astype(jnp.int32)
    zeros_tile = jnp.zeros((_CHUNK, _D), jnp.float32)
    zeros_one = jnp.zeros((_CHUNK,), jnp.float32)

    scatter = functools.partial(
        pl.kernel,
        out_type=[jax.ShapeDtypeStruct((_NC, _NROWS, _D), jnp.float32),
                  jax.ShapeDtypeStruct((_NC * _NROWS,), jnp.float32)],
        mesh=plsc.VectorSubcoreMesh(core_axis_name="c", subcore_axis_name="s"),
        compiler_params=pltpu.CompilerParams(use_tc_tiling_on_sc=True),
        scratch_types=[
            pltpu.VMEM_SHARED((_NROWS, _D), jnp.float32),
            pltpu.VMEM_SHARED((_NROWS,), jnp.float32),
            pltpu.VMEM((_CHUNK,), jnp.int32),
            pltpu.VMEM((_CHUNK,), jnp.int32),
            pltpu.VMEM((_CHUNK,), jnp.int32),
            pltpu.VMEM((_CHUNK,), jnp.int32),
            pltpu.VMEM((_CHUNK,), jnp.int32),
            pltpu.VMEM((_CHUNK,), jnp.int32),
            pltpu.VMEM((_CHUNK, _D), jnp.float32),
            pltpu.VMEM((_CHUNK, _D), jnp.float32),
            pltpu.VMEM((_CHUNK,), jnp.float32),
            pltpu.VMEM((_CHUNK,), jnp.float32),
            pltpu.SemaphoreType.DMA,
            pltpu.SemaphoreType.DMA,
            pltpu.SemaphoreType.DMA,
            pltpu.SemaphoreType.DMA,
            pltpu.SemaphoreType.DMA,
            pltpu.SemaphoreType.DMA,
        ],
    )(_sc_scatter)
    acc, denp = scatter(ei, table, w, zeros_tile, zeros_one)
    dsum = (denp[:_N] + denp[_NROWS:_NROWS + _N] + w).reshape(_N, 1)

    nb = _N // 10
    out = pl.pallas_call(
        _norm_body,
        grid=(10,),
        in_specs=[
            pl.BlockSpec((1, nb, _D), lambda i: (0, i, 0)),
            pl.BlockSpec((1, nb, _D), lambda i: (1, i, 0)),
            pl.BlockSpec((nb, _D), lambda i: (i, 0)),
            pl.BlockSpec((nb, 1), lambda i: (i, 0)),
        ],
        out_specs=pl.BlockSpec((nb, _D), lambda i: (i, 0)),
        out_shape=jax.ShapeDtypeStruct((_N, _D), jnp.float32),
    )(acc, acc, table, dsum)
    return out
